# baseline (speedup 1.0000x reference)
import jax
import jax.numpy as jnp
from jax import lax
from jax.experimental import pallas as pl
from jax.experimental.pallas import tpu as pltpu
from jax.experimental.pallas import tpu_sc as plsc

B, L = 4096, 50
N = B * L
D = 128
NUM_SKU = 100000
NC, NS, LANES = 2, 16, 16
BLANES = 2 * LANES
NW = NC * NS
DW = D // 2
TOK_PER_W = N // NW
CHUNK = 64
NCH = TOK_PER_W // CHUNK
NITER = NCH // 2

BS_BIG = 2000
BB = 256


def _pack_words(p):
    u = lax.bitcast_convert_type(p, jnp.int32)
    rnd = u + 0x7FFF + (lax.shift_right_logical(u, 16) & 1)
    b16 = lax.shift_right_logical(rnd, 16)
    return b16[:, :DW] | lax.shift_left(b16[:, DW:], 16)


def _proj_big_body(sku_ref, word_ref, ws_ref, ww_ref, psku_ref, pword_ref):
    i = pl.program_id(0)
    row0 = lax.broadcasted_iota(jnp.int32, (BS_BIG, 1), 0) + i * BS_BIG
    mask = row0 != 0
    s = jnp.where(mask, sku_ref[...], 0.0)
    w = jnp.where(mask, word_ref[...], 0.0)
    psku_ref[...] = _pack_words(
        jnp.dot(s, ws_ref[...], preferred_element_type=jnp.float32,
                precision=lax.Precision.HIGHEST))
    pword_ref[...] = _pack_words(
        jnp.dot(w, ww_ref[...], preferred_element_type=jnp.float32,
                precision=lax.Precision.HIGHEST))


def _project_big(sku_table, word_table, ws, ww):
    grid = (NUM_SKU // BS_BIG,)
    return pl.pallas_call(
        _proj_big_body,
        grid=grid,
        in_specs=[
            pl.BlockSpec((BS_BIG, 64), lambda i: (i, 0)),
            pl.BlockSpec((BS_BIG, 64), lambda i: (i, 0)),
            pl.BlockSpec((64, D), lambda i: (0, 0)),
            pl.BlockSpec((64, D), lambda i: (0, 0)),
        ],
        out_specs=[
            pl.BlockSpec((BS_BIG, DW), lambda i: (i, 0)),
            pl.BlockSpec((BS_BIG, DW), lambda i: (i, 0)),
        ],
        out_shape=[
            jax.ShapeDtypeStruct((NUM_SKU, DW), jnp.int32),
            jax.ShapeDtypeStruct((NUM_SKU, DW), jnp.int32),
        ],
    )(sku_table, word_table, ws, ww)


def _proj_small_body(ev_ref, ca_ref, pr_ref, we_ref, wc_ref, wp_ref, b_ref,
                     pe_ref, pc_ref, pp_ref):
    def masked(x_ref):
        m = lax.broadcasted_iota(jnp.int32, (x_ref.shape[0], 1), 0) != 0
        return jnp.where(m, x_ref[...], 0.0)

    pe_ref[...] = _pack_words(
        jnp.dot(masked(ev_ref), we_ref[...], preferred_element_type=jnp.float32,
                precision=lax.Precision.HIGHEST) + b_ref[...])
    pc_ref[...] = _pack_words(
        jnp.dot(masked(ca_ref), wc_ref[...], preferred_element_type=jnp.float32,
                precision=lax.Precision.HIGHEST))
    pp_ref[...] = _pack_words(
        jnp.dot(masked(pr_ref), wp_ref[...], preferred_element_type=jnp.float32,
                precision=lax.Precision.HIGHEST))


def _project_small(event_table, cat_table, price_table, we, wc, wp, b):
    return pl.pallas_call(
        _proj_small_body,
        out_shape=[
            jax.ShapeDtypeStruct((event_table.shape[0], DW), jnp.int32),
            jax.ShapeDtypeStruct((cat_table.shape[0], DW), jnp.int32),
            jax.ShapeDtypeStruct((price_table.shape[0], DW), jnp.int32),
        ],
    )(event_table, cat_table, price_table, we, wc, wp, b.reshape(1, D))


def _sc_body(pe, ps, pc, pp, pw, eid, sid, cid, prid, wid, out,
             ix0, ix1, ix2, ix3, ix4, buf, obuf, gsem0, gsem1, osem0, osem1):
    c = lax.axis_index("c")
    s = lax.axis_index("s")
    w = s * NC + c
    base = pl.multiple_of(w * TOK_PER_W, TOK_PER_W)
    tables = (pe, ps, pc, pp, pw)
    ids = (eid, sid, cid, prid, wid)
    idxs = (ix0, ix1, ix2, ix3, ix4)
    gsems = (gsem0, gsem1)
    osems = (osem0, osem1)
    for t in range(5):
        pltpu.sync_copy(ids[t].at[pl.ds(base, TOK_PER_W)], idxs[t])

    def g_descs(p, k):
        off = pl.multiple_of(k * CHUNK, CHUNK)
        return [pltpu.make_async_copy(
            tables[t].at[idxs[t].at[pl.ds(off, CHUNK)]], buf.at[p, t], gsems[p])
            for t in range(5)]

    def o_desc(p, k):
        off = pl.multiple_of(k * CHUNK, CHUNK)
        return pltpu.make_async_copy(
            obuf.at[p], out.at[pl.ds(base + off, CHUNK)], osems[p])

    def start_g(p, k):
        for d in g_descs(p, k):
            d.start()

    def wait_g(p, k):
        for d in g_descs(p, k):
            d.wait()

    zero = jnp.zeros((BLANES,), jnp.bfloat16)

    def compute(p):
        def row(i, carry2):
            for j in range(DW // LANES):
                sl = pl.ds(j * LANES, LANES)
                acc = plsc.bitcast(buf[p, 0, i, sl], jnp.bfloat16)
                for t in range(1, 5):
                    acc = acc + plsc.bitcast(buf[p, t, i, sl], jnp.bfloat16)
                lo, hi = plsc.unpack(jnp.maximum(acc, zero),
                                     format=plsc.PackFormat.INTERLEAVED)
                obuf[p, i, pl.ds(j * LANES, LANES)] = lo
                obuf[p, i, pl.ds(DW + j * LANES, LANES)] = hi
            return carry2
        lax.fori_loop(0, CHUNK, row, 0)

    start_g(0, 0)
    start_g(1, 1)

    def body(kk, carry):
        k0 = kk * 2
        k1 = k0 + 1
        for p, k in ((0, k0), (1, k1)):
            @pl.when(kk > 0)
            def _():
                o_desc(p, k - 2).wait()
            wait_g(p, k)
            compute(p)
            o_desc(p, k).start()

            @pl.when(kk < NITER - 1)
            def _():
                start_g(p, k + 2)
        return carry

    lax.fori_loop(0, NITER, body, 0)
    o_desc(0, NCH - 2).wait()
    o_desc(1, NCH - 1).wait()


def _sc_gather_sum(pe, ps, pc, pp, pw, eid, sid, cid, prid, wid):
    mesh = plsc.VectorSubcoreMesh(core_axis_name="c", subcore_axis_name="s")
    return pl.kernel(
        _sc_body,
        out_type=jax.ShapeDtypeStruct((N, D), jnp.float32),
        mesh=mesh,
        compiler_params=pltpu.CompilerParams(
            use_tc_tiling_on_sc=False, needs_layout_passes=False),
        scratch_types=[
            pltpu.VMEM((TOK_PER_W,), jnp.int32),
            pltpu.VMEM((TOK_PER_W,), jnp.int32),
            pltpu.VMEM((TOK_PER_W,), jnp.int32),
            pltpu.VMEM((TOK_PER_W,), jnp.int32),
            pltpu.VMEM((TOK_PER_W,), jnp.int32),
            pltpu.VMEM((2, 5, CHUNK, DW), jnp.int32),
            pltpu.VMEM((2, CHUNK, D), jnp.float32),
            pltpu.SemaphoreType.DMA,
            pltpu.SemaphoreType.DMA,
            pltpu.SemaphoreType.DMA,
            pltpu.SemaphoreType.DMA,
        ],
    )(pe, ps, pc, pp, pw, eid, sid, cid, prid, wid)


def kernel(event_table, sku_table, cat_table, price_table, word_table, W, b,
           event_id, sku_id, cat_id, price_id, word_ids):
    we, ws, wc, wp, ww = W[0:16], W[16:80], W[80:112], W[112:128], W[128:192]
    psku, pword = _project_big(sku_table, word_table, ws, ww)
    pe, pc, pp = _project_small(event_table, cat_table, price_table, we, wc, wp, b)
    ids = [jnp.reshape(x, (N,)).astype(jnp.int32)
           for x in (event_id, sku_id, cat_id, price_id, word_ids)]
    out = _sc_gather_sum(pe, psku, pc, pp, pword, *ids)
    return out.reshape(B, L, D)


# speedup vs baseline: 1.0649x; 1.0649x over previous
import jax
import jax.numpy as jnp
from jax import lax
from jax.experimental import pallas as pl
from jax.experimental.pallas import tpu as pltpu
from jax.experimental.pallas import tpu_sc as plsc

B, L = 4096, 50
N = B * L
D = 128
NUM_SKU = 100000
NC, NS, LANES = 2, 16, 16
BLANES = 2 * LANES
NW = NC * NS
DW = D // 2
TOK_PER_W = N // NW
CHUNK = 80
NCH = TOK_PER_W // CHUNK
NITER = NCH // 2

BS_BIG = 2000
BB = 256


def _pack_words(p):
    u = lax.bitcast_convert_type(p, jnp.int32)
    rnd = u + 0x7FFF + (lax.shift_right_logical(u, 16) & 1)
    b16 = lax.shift_right_logical(rnd, 16)
    return b16[:, :DW] | lax.shift_left(b16[:, DW:], 16)


def _proj_big_body(sku_ref, word_ref, ws_ref, ww_ref, psku_ref, pword_ref):
    i = pl.program_id(0)
    row0 = lax.broadcasted_iota(jnp.int32, (BS_BIG, 1), 0) + i * BS_BIG
    mask = row0 != 0
    s = jnp.where(mask, sku_ref[...], 0.0)
    w = jnp.where(mask, word_ref[...], 0.0)
    psku_ref[...] = _pack_words(
        jnp.dot(s, ws_ref[...], preferred_element_type=jnp.float32))
    pword_ref[...] = _pack_words(
        jnp.dot(w, ww_ref[...], preferred_element_type=jnp.float32))


def _project_big(sku_table, word_table, ws, ww):
    grid = (NUM_SKU // BS_BIG,)
    return pl.pallas_call(
        _proj_big_body,
        grid=grid,
        in_specs=[
            pl.BlockSpec((BS_BIG, 64), lambda i: (i, 0)),
            pl.BlockSpec((BS_BIG, 64), lambda i: (i, 0)),
            pl.BlockSpec((64, D), lambda i: (0, 0)),
            pl.BlockSpec((64, D), lambda i: (0, 0)),
        ],
        out_specs=[
            pl.BlockSpec((BS_BIG, DW), lambda i: (i, 0)),
            pl.BlockSpec((BS_BIG, DW), lambda i: (i, 0)),
        ],
        out_shape=[
            jax.ShapeDtypeStruct((NUM_SKU, DW), jnp.int32),
            jax.ShapeDtypeStruct((NUM_SKU, DW), jnp.int32),
        ],
    )(sku_table, word_table, ws, ww)


def _proj_small_body(ev_ref, ca_ref, pr_ref, we_ref, wc_ref, wp_ref, b_ref,
                     pe_ref, pc_ref, pp_ref):
    def masked(x_ref):
        m = lax.broadcasted_iota(jnp.int32, (x_ref.shape[0], 1), 0) != 0
        return jnp.where(m, x_ref[...], 0.0)

    pe_ref[...] = _pack_words(
        jnp.dot(masked(ev_ref), we_ref[...], preferred_element_type=jnp.float32) + b_ref[...])
    pc_ref[...] = _pack_words(
        jnp.dot(masked(ca_ref), wc_ref[...], preferred_element_type=jnp.float32))
    pp_ref[...] = _pack_words(
        jnp.dot(masked(pr_ref), wp_ref[...], preferred_element_type=jnp.float32))


def _project_small(event_table, cat_table, price_table, we, wc, wp, b):
    return pl.pallas_call(
        _proj_small_body,
        out_shape=[
            jax.ShapeDtypeStruct((event_table.shape[0], DW), jnp.int32),
            jax.ShapeDtypeStruct((cat_table.shape[0], DW), jnp.int32),
            jax.ShapeDtypeStruct((price_table.shape[0], DW), jnp.int32),
        ],
    )(event_table, cat_table, price_table, we, wc, wp, b.reshape(1, D))


def _sc_body(pe, ps, pc, pp, pw, eid, sid, cid, prid, wid, out,
             ix0, ix1, ix2, ix3, ix4, buf, obuf, gsem0, gsem1, osem0, osem1):
    c = lax.axis_index("c")
    s = lax.axis_index("s")
    w = s * NC + c
    base = pl.multiple_of(w * TOK_PER_W, TOK_PER_W)
    tables = (pe, ps, pc, pp, pw)
    ids = (eid, sid, cid, prid, wid)
    idxs = (ix0, ix1, ix2, ix3, ix4)
    gsems = (gsem0, gsem1)
    osems = (osem0, osem1)
    for t in range(5):
        pltpu.sync_copy(ids[t].at[pl.ds(base, TOK_PER_W)], idxs[t])

    def g_descs(p, k):
        off = pl.multiple_of(k * CHUNK, CHUNK)
        return [pltpu.make_async_copy(
            tables[t].at[idxs[t].at[pl.ds(off, CHUNK)]], buf.at[p, t], gsems[p])
            for t in range(5)]

    def o_desc(p, k):
        off = pl.multiple_of(k * (CHUNK // 2), CHUNK // 2)
        return pltpu.make_async_copy(
            obuf.at[p], out.at[pl.ds(base // 2 + off, CHUNK // 2)], osems[p])

    def start_g(p, k):
        for d in g_descs(p, k):
            d.start()

    def wait_g(p, k):
        for d in g_descs(p, k):
            d.wait()

    zero = jnp.zeros((BLANES,), jnp.bfloat16)

    def compute(p):
        def rowpair(r, carry2):
            for h in range(2):
                i = r * 2 + h
                for j in range(DW // LANES):
                    sl = pl.ds(j * LANES, LANES)
                    acc = plsc.bitcast(buf[p, 0, i, sl], jnp.bfloat16)
                    for t in range(1, 5):
                        acc = acc + plsc.bitcast(buf[p, t, i, sl], jnp.bfloat16)
                    osl = pl.ds(h * DW + j * LANES, LANES)
                    obuf[p, r, osl] = plsc.bitcast(jnp.maximum(acc, zero),
                                                   jnp.int32)
            return carry2
        lax.fori_loop(0, CHUNK // 2, rowpair, 0, unroll=2)

    start_g(0, 0)
    start_g(1, 1)

    def body(kk, carry):
        k0 = kk * 2
        k1 = k0 + 1
        for p, k in ((0, k0), (1, k1)):
            @pl.when(kk > 0)
            def _():
                o_desc(p, k - 2).wait()
            wait_g(p, k)
            compute(p)
            o_desc(p, k).start()

            @pl.when(kk < NITER - 1)
            def _():
                start_g(p, k + 2)
        return carry

    lax.fori_loop(0, NITER, body, 0)
    o_desc(0, NCH - 2).wait()
    o_desc(1, NCH - 1).wait()


def _sc_gather_sum(pe, ps, pc, pp, pw, eid, sid, cid, prid, wid):
    mesh = plsc.VectorSubcoreMesh(core_axis_name="c", subcore_axis_name="s")
    return pl.kernel(
        _sc_body,
        out_type=jax.ShapeDtypeStruct((N // 2, D), jnp.int32),
        mesh=mesh,
        compiler_params=pltpu.CompilerParams(
            use_tc_tiling_on_sc=False, needs_layout_passes=False),
        scratch_types=[
            pltpu.VMEM((TOK_PER_W,), jnp.int32),
            pltpu.VMEM((TOK_PER_W,), jnp.int32),
            pltpu.VMEM((TOK_PER_W,), jnp.int32),
            pltpu.VMEM((TOK_PER_W,), jnp.int32),
            pltpu.VMEM((TOK_PER_W,), jnp.int32),
            pltpu.VMEM((2, 5, CHUNK, DW), jnp.int32),
            pltpu.VMEM((2, CHUNK // 2, D), jnp.int32),
            pltpu.SemaphoreType.DMA,
            pltpu.SemaphoreType.DMA,
            pltpu.SemaphoreType.DMA,
            pltpu.SemaphoreType.DMA,
        ],
    )(pe, ps, pc, pp, pw, eid, sid, cid, prid, wid)


def _unpack_body(w_ref, out_ref):
    w = w_ref[...]
    lo_f = lax.bitcast_convert_type(lax.shift_left(w, 16), jnp.float32)
    hi_f = lax.bitcast_convert_type(w & jnp.int32(-65536), jnp.float32)
    even = jnp.concatenate([lo_f[:, :DW], hi_f[:, :DW]], axis=1)
    odd = jnp.concatenate([lo_f[:, DW:], hi_f[:, DW:]], axis=1)
    x = jnp.stack([even, odd], axis=1).reshape(BB * L, D)
    out_ref[...] = x.reshape(BB, L, D)


def _unpack(out_words):
    grid = (B // BB,)
    return pl.pallas_call(
        _unpack_body,
        grid=grid,
        in_specs=[pl.BlockSpec((BB * L // 2, D), lambda i: (i, 0))],
        out_specs=pl.BlockSpec((BB, L, D), lambda i: (i, 0, 0)),
        out_shape=jax.ShapeDtypeStruct((B, L, D), jnp.float32),
    )(out_words)


def kernel(event_table, sku_table, cat_table, price_table, word_table, W, b,
           event_id, sku_id, cat_id, price_id, word_ids):
    we, ws, wc, wp, ww = W[0:16], W[16:80], W[80:112], W[112:128], W[128:192]
    psku, pword = _project_big(sku_table, word_table, ws, ww)
    pe, pc, pp = _project_small(event_table, cat_table, price_table, we, wc, wp, b)
    ids = [jnp.reshape(x, (N,)).astype(jnp.int32)
           for x in (event_id, sku_id, cat_id, price_id, word_ids)]
    out_words = _sc_gather_sum(pe, psku, pc, pp, pword, *ids)
    return _unpack(out_words)


# speedup vs baseline: 1.0852x; 1.0191x over previous
import jax
import jax.numpy as jnp
from jax import lax
from jax.experimental import pallas as pl
from jax.experimental.pallas import tpu as pltpu
from jax.experimental.pallas import tpu_sc as plsc

B, L = 4096, 50
N = B * L
D = 128
NUM_SKU = 100000
NC, NS, LANES = 2, 16, 16
BLANES = 2 * LANES
NW = NC * NS
DW = D // 2
TOK_PER_W = N // NW
CHUNK = 80
NCH = TOK_PER_W // CHUNK
NITER = NCH // 2

BS_BIG = 4000
BB = 256


def _pack_words(p):
    u = lax.bitcast_convert_type(p, jnp.int32)
    rnd = u + 0x7FFF + (lax.shift_right_logical(u, 16) & 1)
    b16 = lax.shift_right_logical(rnd, 16)
    return b16[:, :DW] | lax.shift_left(b16[:, DW:], 16)


def _proj_big_body(sku_ref, word_ref, ws_ref, ww_ref, psku_ref, pword_ref):
    i = pl.program_id(0)
    row0 = lax.broadcasted_iota(jnp.int32, (BS_BIG, 1), 0) + i * BS_BIG
    mask = row0 != 0
    s = jnp.where(mask, sku_ref[...], 0.0)
    w = jnp.where(mask, word_ref[...], 0.0)
    psku_ref[...] = _pack_words(
        jnp.dot(s, ws_ref[...], preferred_element_type=jnp.float32))
    pword_ref[...] = _pack_words(
        jnp.dot(w, ww_ref[...], preferred_element_type=jnp.float32))


def _project_big(sku_table, word_table, ws, ww):
    grid = (NUM_SKU // BS_BIG,)
    return pl.pallas_call(
        _proj_big_body,
        grid=grid,
        in_specs=[
            pl.BlockSpec((BS_BIG, 64), lambda i: (i, 0)),
            pl.BlockSpec((BS_BIG, 64), lambda i: (i, 0)),
            pl.BlockSpec((64, D), lambda i: (0, 0)),
            pl.BlockSpec((64, D), lambda i: (0, 0)),
        ],
        out_specs=[
            pl.BlockSpec((BS_BIG, DW), lambda i: (i, 0)),
            pl.BlockSpec((BS_BIG, DW), lambda i: (i, 0)),
        ],
        out_shape=[
            jax.ShapeDtypeStruct((NUM_SKU, DW), jnp.int32),
            jax.ShapeDtypeStruct((NUM_SKU, DW), jnp.int32),
        ],
    )(sku_table, word_table, ws, ww)


def _proj_small_body(ev_ref, ca_ref, pr_ref, we_ref, wc_ref, wp_ref, b_ref,
                     pe_ref, pc_ref, pp_ref):
    def masked(x_ref):
        m = lax.broadcasted_iota(jnp.int32, (x_ref.shape[0], 1), 0) != 0
        return jnp.where(m, x_ref[...], 0.0)

    pe_ref[...] = _pack_words(
        jnp.dot(masked(ev_ref), we_ref[...], preferred_element_type=jnp.float32) + b_ref[...])
    pc_ref[...] = _pack_words(
        jnp.dot(masked(ca_ref), wc_ref[...], preferred_element_type=jnp.float32))
    pp_ref[...] = _pack_words(
        jnp.dot(masked(pr_ref), wp_ref[...], preferred_element_type=jnp.float32))


def _project_small(event_table, cat_table, price_table, we, wc, wp, b):
    return pl.pallas_call(
        _proj_small_body,
        out_shape=[
            jax.ShapeDtypeStruct((event_table.shape[0], DW), jnp.int32),
            jax.ShapeDtypeStruct((cat_table.shape[0], DW), jnp.int32),
            jax.ShapeDtypeStruct((price_table.shape[0], DW), jnp.int32),
        ],
    )(event_table, cat_table, price_table, we, wc, wp, b.reshape(1, D))


def _sc_body(pe, ps, pc, pp, pw, eid, sid, cid, prid, wid, out,
             ix0, ix1, ix2, ix3, ix4, buf, obuf,
             gsem0, gsem1, osem0, osem1):
    c = lax.axis_index("c")
    s = lax.axis_index("s")
    w = s * NC + c
    base = pl.multiple_of(w * TOK_PER_W, TOK_PER_W)
    tables = (pe, ps, pc, pp, pw)
    ids = (eid, sid, cid, prid, wid)
    idxs = (ix0, ix1, ix2, ix3, ix4)
    gsems = (gsem0, gsem1)
    osems = (osem0, osem1)
    for t in range(5):
        pltpu.sync_copy(ids[t].at[pl.ds(base, TOK_PER_W)], idxs[t])

    def g_descs(p, k):
        off = pl.multiple_of(k * CHUNK, CHUNK)
        return [pltpu.make_async_copy(
            tables[t].at[idxs[t].at[pl.ds(off, CHUNK)]], buf.at[p, t], gsems[p])
            for t in range(5)]

    def o_desc(p, k):
        off = pl.multiple_of(k * (CHUNK // 2), CHUNK // 2)
        return pltpu.make_async_copy(
            obuf.at[p], out.at[pl.ds(base // 2 + off, CHUNK // 2)], osems[p])

    def start_g(p, k):
        for d in g_descs(p, k):
            d.start()

    def wait_g(p, k):
        for d in g_descs(p, k):
            d.wait()

    zero = jnp.zeros((BLANES,), jnp.bfloat16)

    def compute(p):
        def rowpair(r, carry2):
            for h in range(2):
                i = r * 2 + h
                for j in range(DW // LANES):
                    sl = pl.ds(j * LANES, LANES)
                    acc = plsc.bitcast(buf[p, 0, i, sl], jnp.bfloat16)
                    for t in range(1, 5):
                        acc = acc + plsc.bitcast(buf[p, t, i, sl], jnp.bfloat16)
                    osl = pl.ds(h * DW + j * LANES, LANES)
                    obuf[p, r, osl] = plsc.bitcast(jnp.maximum(acc, zero),
                                                   jnp.int32)
            return carry2
        lax.fori_loop(0, CHUNK // 2, rowpair, 0, unroll=4)

    start_g(0, 0)
    start_g(1, 1)

    def body(kk, carry):
        k0 = kk * 2
        k1 = k0 + 1
        for p, k in ((0, k0), (1, k1)):
            @pl.when(kk > 0)
            def _():
                o_desc(p, k - 2).wait()
            wait_g(p, k)
            compute(p)
            o_desc(p, k).start()

            @pl.when(kk < NITER - 1)
            def _():
                start_g(p, k + 2)
        return carry

    lax.fori_loop(0, NITER, body, 0)
    o_desc(0, NCH - 2).wait()
    o_desc(1, NCH - 1).wait()


def _sc_gather_sum(pe, ps, pc, pp, pw, eid, sid, cid, prid, wid):
    mesh = plsc.VectorSubcoreMesh(core_axis_name="c", subcore_axis_name="s")
    return pl.kernel(
        _sc_body,
        out_type=jax.ShapeDtypeStruct((N // 2, D), jnp.int32),
        mesh=mesh,
        compiler_params=pltpu.CompilerParams(
            use_tc_tiling_on_sc=False, needs_layout_passes=False),
        scratch_types=[
            pltpu.VMEM((TOK_PER_W,), jnp.int32),
            pltpu.VMEM((TOK_PER_W,), jnp.int32),
            pltpu.VMEM((TOK_PER_W,), jnp.int32),
            pltpu.VMEM((TOK_PER_W,), jnp.int32),
            pltpu.VMEM((TOK_PER_W,), jnp.int32),
            pltpu.VMEM((2, 5, CHUNK, DW), jnp.int32),
            pltpu.VMEM((2, CHUNK // 2, D), jnp.int32),
            pltpu.SemaphoreType.DMA,
            pltpu.SemaphoreType.DMA,
            pltpu.SemaphoreType.DMA,
            pltpu.SemaphoreType.DMA,
        ],
    )(pe, ps, pc, pp, pw, eid, sid, cid, prid, wid)


def _unpack_body(w_ref, out_ref):
    w = w_ref[...]
    lo_f = lax.bitcast_convert_type(lax.shift_left(w, 16), jnp.float32)
    hi_f = lax.bitcast_convert_type(w & jnp.int32(-65536), jnp.float32)
    even = jnp.concatenate([lo_f[:, :DW], hi_f[:, :DW]], axis=1)
    odd = jnp.concatenate([lo_f[:, DW:], hi_f[:, DW:]], axis=1)
    x = jnp.stack([even, odd], axis=1).reshape(BB * L, D)
    out_ref[...] = x.reshape(BB, L, D)


def _unpack(out_words):
    grid = (B // BB,)
    return pl.pallas_call(
        _unpack_body,
        grid=grid,
        in_specs=[pl.BlockSpec((BB * L // 2, D), lambda i: (i, 0))],
        out_specs=pl.BlockSpec((BB, L, D), lambda i: (i, 0, 0)),
        out_shape=jax.ShapeDtypeStruct((B, L, D), jnp.float32),
    )(out_words)


def kernel(event_table, sku_table, cat_table, price_table, word_table, W, b,
           event_id, sku_id, cat_id, price_id, word_ids):
    we, ws, wc, wp, ww = W[0:16], W[16:80], W[80:112], W[112:128], W[128:192]
    psku, pword = _project_big(sku_table, word_table, ws, ww)
    pe, pc, pp = _project_small(event_table, cat_table, price_table, we, wc, wp, b)
    ids = [jnp.reshape(x, (N,)).astype(jnp.int32)
           for x in (event_id, sku_id, cat_id, price_id, word_ids)]
    out_words = _sc_gather_sum(pe, psku, pc, pp, pword, *ids)
    return _unpack(out_words)
